# baseline (speedup 1.0000x reference)
import functools

import jax
import jax.numpy as jnp
from jax import lax
from jax.experimental import pallas as pl
from jax.experimental.pallas import tpu as pltpu
from jax.experimental.pallas import tpu_sc as plsc

VOCAB = 100000
EMB_IN = 2048
EMB_OUT = 1024
BATCH = 16384

NUM_CORES = 2
NUM_SUBCORES = 16
NUM_WORKERS = NUM_CORES * NUM_SUBCORES
CHUNK = 16
PIPE_SIZES = (10240, 6144)


def _make_sc_body(b_per_w, n_chunks):
    def _sc_gather_body(table_hbm, idx_hbm, out_hbm, idx_v, buf0, buf1,
                        sem0, sem1):
        wid = lax.axis_index("s") * NUM_CORES + lax.axis_index("c")
        base = pl.multiple_of(wid * b_per_w, 8)
        pltpu.sync_copy(idx_hbm.at[pl.ds(base, b_per_w)], idx_v)
        bufs = (buf0, buf1)
        sems = (sem0, sem1)

        def start(c):
            return pltpu.async_copy(
                table_hbm.at[idx_v.at[pl.ds(c * CHUNK, CHUNK)]],
                bufs[c % 2],
                sems[c % 2],
            )

        pending = start(0)
        for c in range(n_chunks):
            pending.wait()
            if c + 1 < n_chunks:
                pending = start(c + 1)
            pltpu.sync_copy(
                bufs[c % 2], out_hbm.at[pl.ds(base + c * CHUNK, CHUNK)]
            )

    return _sc_gather_body


@functools.lru_cache(maxsize=None)
def _sc_gather(rows):
    b_per_w = rows // NUM_WORKERS
    n_chunks = b_per_w // CHUNK
    return pl.kernel(
        _make_sc_body(b_per_w, n_chunks),
        mesh=plsc.VectorSubcoreMesh(core_axis_name="c", subcore_axis_name="s"),
        out_type=jax.ShapeDtypeStruct((rows, EMB_IN), jnp.float32),
        scratch_types=[
            pltpu.VMEM((b_per_w,), jnp.int32),
            pltpu.VMEM((CHUNK, EMB_IN), jnp.float32),
            pltpu.VMEM((CHUNK, EMB_IN), jnp.float32),
            pltpu.SemaphoreType.DMA,
            pltpu.SemaphoreType.DMA,
        ],
    )


BM = 1024


def _mm_body(emb_ref, w_ref, b_ref, out_ref):
    h = jnp.tanh(emb_ref[...]).astype(jnp.bfloat16)
    wb = w_ref[...].astype(jnp.bfloat16)
    out_ref[...] = (
        jnp.dot(h, wb, preferred_element_type=jnp.float32) + b_ref[...]
    )


def _mm_body_alias(emb_ref, w_ref, b_ref, prev_ref, out_ref):
    del prev_ref
    _mm_body(emb_ref, w_ref, b_ref, out_ref)


def _tc_matmul_slice(emb, w, b2d, prev, row_base):
    blocks_per_call = emb.shape[0] // BM
    base = row_base // BM
    out_shape = jax.ShapeDtypeStruct((BATCH, EMB_OUT), jnp.float32)
    common_specs = [
        pl.BlockSpec((BM, EMB_IN), lambda i: (i, 0)),
        pl.BlockSpec((EMB_IN, EMB_OUT), lambda i: (0, 0)),
        pl.BlockSpec((1, EMB_OUT), lambda i: (0, 0)),
    ]
    out_spec = pl.BlockSpec((BM, EMB_OUT), lambda i: (base + i, 0))
    if prev is None:
        return pl.pallas_call(
            _mm_body,
            grid=(blocks_per_call,),
            in_specs=common_specs,
            out_specs=out_spec,
            out_shape=out_shape,
        )(emb, w, b2d)
    return pl.pallas_call(
        _mm_body_alias,
        grid=(blocks_per_call,),
        in_specs=common_specs + [pl.BlockSpec(memory_space=pl.ANY)],
        out_specs=out_spec,
        out_shape=out_shape,
        input_output_aliases={3: 0},
    )(emb, w, b2d, prev)


def kernel(attr, table, W, b):
    idx = attr.reshape(-1)
    b2d = b.reshape(1, EMB_OUT)
    bases = []
    embs = []
    off = 0
    for rows in PIPE_SIZES:
        idx_k = lax.slice(idx, (off,), (off + rows,))
        embs.append(_sc_gather(rows)(table, idx_k))
        bases.append(off)
        off += rows
    out = None
    for emb_k, row_base in zip(embs, bases):
        out = _tc_matmul_slice(emb_k, W, b2d, out, row_base)
    return out


# speedup vs baseline: 1.0244x; 1.0244x over previous
import functools

import jax
import jax.numpy as jnp
from jax import lax
from jax.experimental import pallas as pl
from jax.experimental.pallas import tpu as pltpu
from jax.experimental.pallas import tpu_sc as plsc

VOCAB = 100000
EMB_IN = 2048
EMB_OUT = 1024
BATCH = 16384

NUM_CORES = 2
NUM_SUBCORES = 16
NUM_WORKERS = NUM_CORES * NUM_SUBCORES
CHUNK = 16
PIPE_SIZES = (8192, 8192)


def _make_sc_body(b_per_w, n_chunks):
    def _sc_gather_body(table_hbm, idx_hbm, out_hbm, idx_v, buf0, buf1,
                        sem0, sem1):
        wid = lax.axis_index("s") * NUM_CORES + lax.axis_index("c")
        base = pl.multiple_of(wid * b_per_w, 8)
        pltpu.sync_copy(idx_hbm.at[pl.ds(base, b_per_w)], idx_v)
        bufs = (buf0, buf1)
        sems = (sem0, sem1)

        def start(c):
            return pltpu.async_copy(
                table_hbm.at[idx_v.at[pl.ds(c * CHUNK, CHUNK)]],
                bufs[c % 2],
                sems[c % 2],
            )

        pending = start(0)
        for c in range(n_chunks):
            pending.wait()
            if c + 1 < n_chunks:
                pending = start(c + 1)
            pltpu.sync_copy(
                bufs[c % 2], out_hbm.at[pl.ds(base + c * CHUNK, CHUNK)]
            )

    return _sc_gather_body


@functools.lru_cache(maxsize=None)
def _sc_gather(rows):
    b_per_w = rows // NUM_WORKERS
    n_chunks = b_per_w // CHUNK
    return pl.kernel(
        _make_sc_body(b_per_w, n_chunks),
        mesh=plsc.VectorSubcoreMesh(core_axis_name="c", subcore_axis_name="s"),
        out_type=jax.ShapeDtypeStruct((rows, EMB_IN), jnp.float32),
        scratch_types=[
            pltpu.VMEM((b_per_w,), jnp.int32),
            pltpu.VMEM((CHUNK, EMB_IN), jnp.float32),
            pltpu.VMEM((CHUNK, EMB_IN), jnp.float32),
            pltpu.SemaphoreType.DMA,
            pltpu.SemaphoreType.DMA,
        ],
    )


BM = 1024


def _mm_body(emb_ref, w_ref, b_ref, out_ref):
    h = jnp.tanh(emb_ref[...]).astype(jnp.bfloat16)
    wb = w_ref[...].astype(jnp.bfloat16)
    out_ref[...] = (
        jnp.dot(h, wb, preferred_element_type=jnp.float32) + b_ref[...]
    )


def _mm_body_alias(emb_ref, w_ref, b_ref, prev_ref, out_ref):
    del prev_ref
    _mm_body(emb_ref, w_ref, b_ref, out_ref)


def _tc_matmul_slice(emb, w, b2d, prev, row_base):
    blocks_per_call = emb.shape[0] // BM
    base = row_base // BM
    out_shape = jax.ShapeDtypeStruct((BATCH, EMB_OUT), jnp.float32)
    common_specs = [
        pl.BlockSpec((BM, EMB_IN), lambda i: (i, 0)),
        pl.BlockSpec((EMB_IN, EMB_OUT), lambda i: (0, 0)),
        pl.BlockSpec((1, EMB_OUT), lambda i: (0, 0)),
    ]
    out_spec = pl.BlockSpec((BM, EMB_OUT), lambda i: (base + i, 0))
    if prev is None:
        return pl.pallas_call(
            _mm_body,
            grid=(blocks_per_call,),
            in_specs=common_specs,
            out_specs=out_spec,
            out_shape=out_shape,
        )(emb, w, b2d)
    return pl.pallas_call(
        _mm_body_alias,
        grid=(blocks_per_call,),
        in_specs=common_specs + [pl.BlockSpec(memory_space=pl.ANY)],
        out_specs=out_spec,
        out_shape=out_shape,
        input_output_aliases={3: 0},
    )(emb, w, b2d, prev)


def kernel(attr, table, W, b):
    idx = attr.reshape(-1)
    b2d = b.reshape(1, EMB_OUT)
    bases = []
    embs = []
    off = 0
    for rows in PIPE_SIZES:
        idx_k = lax.slice(idx, (off,), (off + rows,))
        embs.append(_sc_gather(rows)(table, idx_k))
        bases.append(off)
        off += rows
    out = None
    for emb_k, row_base in zip(embs, bases):
        out = _tc_matmul_slice(emb_k, W, b2d, out, row_base)
    return out
